# Initial kernel scaffold; baseline (speedup 1.0000x reference)
import jax
import jax.numpy as jnp
from jax.experimental import pallas as pl


def kernel(img, w0, b0, w1, b1, w2, b2, w3, b3, w4, b4):
    raise NotImplementedError("write your pallas kernel here")


# speedup vs baseline: 9.3293x; 9.3293x over previous
import jax
import jax.numpy as jnp
from jax.experimental import pallas as pl
from jax.experimental.pallas import tpu as pltpu

_EPS = 1e-5
_SLOPE = 0.2


def _lrelu(x):
    return jnp.where(x >= 0.0, x, _SLOPE * x)


def _instnorm_lrelu(x):
    inv = 1.0 / x.shape[0]
    mean = jnp.sum(x, axis=0, keepdims=True) * inv
    xc = x - mean
    var = jnp.sum(xc * xc, axis=0, keepdims=True) * inv
    return _lrelu(xc * jax.lax.rsqrt(var + _EPS))


def _pad_hw(x):
    H, W, C = x.shape
    zc = jnp.zeros((H, 1, C), x.dtype)
    x = jnp.concatenate([zc, x, zc], axis=1)
    zr = jnp.zeros((1, W + 2, C), x.dtype)
    return jnp.concatenate([zr, x, zr], axis=0)


def _conv_taps1(xp, w_ref, bias, ho, wo):
    acc = bias
    for kh in range(3):
        for kw in range(3):
            a = xp[kh:kh + ho, kw:kw + wo, :].reshape(ho * wo, xp.shape[2])
            acc = acc + jnp.dot(a, w_ref[kh * 3 + kw],
                                preferred_element_type=jnp.float32)
    return acc


def _conv_taps2(s_ref, w_ref, bias, ho, wo):
    acc = bias
    c = s_ref.shape[2]
    for kh in range(3):
        for kw in range(3):
            a = s_ref[pl.Slice(kh, ho, 2), pl.Slice(kw, wo, 2), :]
            a = a.reshape(ho * wo, c).astype(jnp.bfloat16)
            acc = acc + jnp.dot(a, w_ref[kh * 3 + kw],
                                preferred_element_type=jnp.float32)
    return acc


def _disc_kernel(a0_ref, w0_ref, b0_ref, w1_ref, b1_ref, w2_ref, b2_ref,
                 w3_ref, b3_ref, w4_ref, b4_ref, o_ref, s1_ref, s2_ref):
    x = jnp.dot(a0_ref[0], w0_ref[...], preferred_element_type=jnp.float32)
    x = _lrelu(x + b0_ref[...])
    s1_ref[...] = _pad_hw(x.reshape(128, 128, 64))

    x = _conv_taps2(s1_ref, w1_ref, b1_ref[...], 64, 64)
    x = _instnorm_lrelu(x)
    s2_ref[...] = _pad_hw(x.reshape(64, 64, 128))

    x = _conv_taps2(s2_ref, w2_ref, b2_ref[...], 32, 32)
    x = _instnorm_lrelu(x).astype(jnp.bfloat16)
    xp = _pad_hw(x.reshape(32, 32, 256))

    x = _conv_taps1(xp, w3_ref, b3_ref[...], 32, 32)
    x = _instnorm_lrelu(x).astype(jnp.bfloat16)
    xp = _pad_hw(x.reshape(32, 32, 512))

    o_ref[0] = _conv_taps1(xp, w4_ref, b4_ref[...], 32, 32)


def _im2col_l0(x):
    N = x.shape[0]
    xp = jnp.pad(x, ((0, 0), (1, 1), (1, 1), (0, 0))).astype(jnp.bfloat16)
    cols = []
    for kh in range(3):
        for kw in range(3):
            cols.append(xp[:, kh:kh + 255:2, kw:kw + 255:2, :])
    return jnp.stack(cols, axis=3).reshape(N, 128 * 128, 27)


def kernel(img, w0, b0, w1, b1, w2, b2, w3, b3, w4, b4):
    N = img.shape[0]
    x = jnp.transpose(img, (0, 2, 3, 1)).astype(jnp.float32)
    a0 = _im2col_l0(x)

    w0m = w0.reshape(27, 64).astype(jnp.bfloat16)
    b0m = b0.reshape(1, 64).astype(jnp.float32)
    w1m = w1.reshape(9, 64, 128).astype(jnp.bfloat16)
    b1m = b1.reshape(1, 128).astype(jnp.float32)
    w2m = w2.reshape(9, 128, 256).astype(jnp.bfloat16)
    b2m = b2.reshape(1, 256).astype(jnp.float32)
    w3m = w3.reshape(9, 256, 512).astype(jnp.bfloat16)
    b3m = b3.reshape(1, 512).astype(jnp.float32)
    w4m = jnp.pad(w4.reshape(9, 512, 1), ((0, 0), (0, 0), (0, 127)))
    w4m = w4m.astype(jnp.bfloat16)
    b4m = jnp.pad(b4.reshape(1, 1), ((0, 0), (0, 127))).astype(jnp.float32)

    out = pl.pallas_call(
        _disc_kernel,
        out_shape=jax.ShapeDtypeStruct((N, 1024, 128), jnp.float32),
        grid=(N,),
        in_specs=[
            pl.BlockSpec((1, 16384, 27), lambda n: (n, 0, 0)),
            pl.BlockSpec((27, 64), lambda n: (0, 0)),
            pl.BlockSpec((1, 64), lambda n: (0, 0)),
            pl.BlockSpec((9, 64, 128), lambda n: (0, 0, 0)),
            pl.BlockSpec((1, 128), lambda n: (0, 0)),
            pl.BlockSpec((9, 128, 256), lambda n: (0, 0, 0)),
            pl.BlockSpec((1, 256), lambda n: (0, 0)),
            pl.BlockSpec((9, 256, 512), lambda n: (0, 0, 0)),
            pl.BlockSpec((1, 512), lambda n: (0, 0)),
            pl.BlockSpec((9, 512, 128), lambda n: (0, 0, 0)),
            pl.BlockSpec((1, 128), lambda n: (0, 0)),
        ],
        out_specs=pl.BlockSpec((1, 1024, 128), lambda n: (n, 0, 0)),
        scratch_shapes=[
            pltpu.VMEM((130, 130, 64), jnp.float32),
            pltpu.VMEM((66, 66, 128), jnp.float32),
        ],
        compiler_params=pltpu.CompilerParams(
            dimension_semantics=("parallel",),
            vmem_limit_bytes=48 * 1024 * 1024,
        ),
    )(a0, w0m, b0m, w1m, b1m, w2m, b2m, w3m, b3m, w4m, b4m)

    return out[:, :, 0].reshape(N, 1, 32, 32)


# speedup vs baseline: 9.8546x; 1.0563x over previous
import jax
import jax.numpy as jnp
from jax.experimental import pallas as pl
from jax.experimental.pallas import tpu as pltpu

_EPS = 1e-5
_SLOPE = 0.2


def _lrelu(x):
    return jnp.where(x >= 0.0, x, _SLOPE * x)


def _instnorm_lrelu(x):
    inv = 1.0 / x.shape[0]
    mean = jnp.sum(x, axis=0, keepdims=True) * inv
    xc = x - mean
    var = jnp.sum(xc * xc, axis=0, keepdims=True) * inv
    return _lrelu(xc * jax.lax.rsqrt(var + _EPS))


def _pad_hw(x):
    H, W, C = x.shape
    zc = jnp.zeros((H, 1, C), x.dtype)
    x = jnp.concatenate([zc, x, zc], axis=1)
    zr = jnp.zeros((1, W + 2, C), x.dtype)
    return jnp.concatenate([zr, x, zr], axis=0)


def _conv_taps1(xp, w_ref, bias, ho, wo):
    acc = bias
    for kh in range(3):
        for kw in range(3):
            a = xp[kh:kh + ho, kw:kw + wo, :].reshape(ho * wo, xp.shape[2])
            acc = acc + jnp.dot(a, w_ref[kh * 3 + kw],
                                preferred_element_type=jnp.float32)
    return acc


def _conv_taps2(s_ref, w_ref, bias, ho, wo):
    acc = bias
    c = s_ref.shape[2]
    for kh in range(3):
        for kw in range(3):
            a = s_ref[pl.Slice(kh, ho, 2), pl.Slice(kw, wo, 2), :]
            a = a.reshape(ho * wo, c).astype(jnp.bfloat16)
            acc = acc + jnp.dot(a, w_ref[kh * 3 + kw],
                                preferred_element_type=jnp.float32)
    return acc


def _disc_kernel(a0_ref, w0_ref, b0_ref, w1_ref, b1_ref, w2_ref, b2_ref,
                 w3_ref, b3_ref, w4_ref, b4_ref, o_ref, s1_ref, s2_ref):
    x = jax.lax.dot_general(a0_ref[0], w0_ref[...],
                            (((0,), (0,)), ((), ())),
                            preferred_element_type=jnp.float32)
    x = _lrelu(x + b0_ref[...])
    s1_ref[...] = _pad_hw(x.reshape(128, 128, 64))

    x = _conv_taps2(s1_ref, w1_ref, b1_ref[...], 64, 64)
    x = _instnorm_lrelu(x)
    s2_ref[...] = _pad_hw(x.reshape(64, 64, 128))

    x = _conv_taps2(s2_ref, w2_ref, b2_ref[...], 32, 32)
    x = _instnorm_lrelu(x).astype(jnp.bfloat16)
    xp = _pad_hw(x.reshape(32, 32, 256))

    x = _conv_taps1(xp, w3_ref, b3_ref[...], 32, 32)
    x = _instnorm_lrelu(x).astype(jnp.bfloat16)
    xp = _pad_hw(x.reshape(32, 32, 512))

    o_ref[0] = _conv_taps1(xp, w4_ref, b4_ref[...], 32, 32)


def _im2col_l0(img):
    N = img.shape[0]
    xp = jnp.pad(img, ((0, 0), (0, 0), (1, 1), (1, 1))).astype(jnp.bfloat16)
    cols = []
    for kh in range(3):
        for kw in range(3):
            cols.append(xp[:, :, kh:kh + 255:2, kw:kw + 255:2])
    return jnp.stack(cols, axis=2).reshape(N, 27, 128 * 128)


def kernel(img, w0, b0, w1, b1, w2, b2, w3, b3, w4, b4):
    N = img.shape[0]
    a0 = _im2col_l0(img)

    w0m = jnp.transpose(w0, (2, 0, 1, 3)).reshape(27, 64).astype(jnp.bfloat16)
    b0m = b0.reshape(1, 64).astype(jnp.float32)
    w1m = w1.reshape(9, 64, 128).astype(jnp.bfloat16)
    b1m = b1.reshape(1, 128).astype(jnp.float32)
    w2m = w2.reshape(9, 128, 256).astype(jnp.bfloat16)
    b2m = b2.reshape(1, 256).astype(jnp.float32)
    w3m = w3.reshape(9, 256, 512).astype(jnp.bfloat16)
    b3m = b3.reshape(1, 512).astype(jnp.float32)
    w4m = jnp.pad(w4.reshape(9, 512, 1), ((0, 0), (0, 0), (0, 127)))
    w4m = w4m.astype(jnp.bfloat16)
    b4m = jnp.pad(b4.reshape(1, 1), ((0, 0), (0, 127))).astype(jnp.float32)

    out = pl.pallas_call(
        _disc_kernel,
        out_shape=jax.ShapeDtypeStruct((N, 1024, 128), jnp.float32),
        grid=(N,),
        in_specs=[
            pl.BlockSpec((1, 27, 16384), lambda n: (n, 0, 0)),
            pl.BlockSpec((27, 64), lambda n: (0, 0)),
            pl.BlockSpec((1, 64), lambda n: (0, 0)),
            pl.BlockSpec((9, 64, 128), lambda n: (0, 0, 0)),
            pl.BlockSpec((1, 128), lambda n: (0, 0)),
            pl.BlockSpec((9, 128, 256), lambda n: (0, 0, 0)),
            pl.BlockSpec((1, 256), lambda n: (0, 0)),
            pl.BlockSpec((9, 256, 512), lambda n: (0, 0, 0)),
            pl.BlockSpec((1, 512), lambda n: (0, 0)),
            pl.BlockSpec((9, 512, 128), lambda n: (0, 0, 0)),
            pl.BlockSpec((1, 128), lambda n: (0, 0)),
        ],
        out_specs=pl.BlockSpec((1, 1024, 128), lambda n: (n, 0, 0)),
        scratch_shapes=[
            pltpu.VMEM((130, 130, 64), jnp.float32),
            pltpu.VMEM((66, 66, 128), jnp.float32),
        ],
        compiler_params=pltpu.CompilerParams(
            dimension_semantics=("parallel",),
            vmem_limit_bytes=48 * 1024 * 1024,
        ),
    )(a0, w0m, b0m, w1m, b1m, w2m, b2m, w3m, b3m, w4m, b4m)

    return out[:, :, 0].reshape(N, 1, 32, 32)


# speedup vs baseline: 27.4748x; 2.7880x over previous
import jax
import jax.numpy as jnp
from jax.experimental import pallas as pl
from jax.experimental.pallas import tpu as pltpu

_EPS = 1e-5
_SLOPE = 0.2


def _lrelu(x):
    return jnp.where(x >= 0.0, x, _SLOPE * x)


def _instnorm_lrelu(x):
    inv = 1.0 / x.shape[0]
    mean = jnp.sum(x, axis=0, keepdims=True) * inv
    xc = x - mean
    var = jnp.sum(xc * xc, axis=0, keepdims=True) * inv
    return _lrelu(xc * jax.lax.rsqrt(var + _EPS))


def _pad_hw(x):
    H, W, C = x.shape
    zc = jnp.zeros((H, 1, C), x.dtype)
    x = jnp.concatenate([zc, x, zc], axis=1)
    zr = jnp.zeros((1, W + 2, C), x.dtype)
    return jnp.concatenate([zr, x, zr], axis=0)


def _conv_taps1(xp, w_ref, bias, ho, wo):
    acc = bias
    for kh in range(3):
        for kw in range(3):
            a = xp[kh:kh + ho, kw:kw + wo, :].reshape(ho * wo, xp.shape[2])
            acc = acc + jnp.dot(a, w_ref[kh * 3 + kw],
                                preferred_element_type=jnp.float32)
    return acc


def _conv_taps2(s_ref, w_ref, bias, ho, wo):
    acc = bias
    c = s_ref.shape[2]
    for kh in range(3):
        for kw in range(3):
            a = s_ref[pl.Slice(kh, ho, 2), pl.Slice(kw, wo, 2), :]
            a = a.reshape(ho * wo, c).astype(jnp.bfloat16)
            acc = acc + jnp.dot(a, w_ref[kh * 3 + kw],
                                preferred_element_type=jnp.float32)
    return acc


def _disc_kernel(imgi_ref, w0_ref, b0_ref, w1_ref, b1_ref, w2_ref, b2_ref,
                 w3_ref, b3_ref, w4_ref, b4_ref, o_ref, s1_ref, s2_ref,
                 se_ref, so_ref, so0_ref):
    v = imgi_ref[0]
    xe = pltpu.unpack_elementwise(v, index=0, packed_dtype=jnp.bfloat16,
                                  unpacked_dtype=jnp.float32)
    xo = pltpu.unpack_elementwise(v, index=1, packed_dtype=jnp.bfloat16,
                                  unpacked_dtype=jnp.float32)
    z1 = jnp.zeros((3, 1, 128), jnp.float32)
    se_ref[...] = jnp.concatenate([z1, xe], axis=1)
    so_ref[...] = jnp.concatenate([z1, xo], axis=1)
    zc = jnp.zeros((3, 256, 1), jnp.float32)
    xo_sh = jnp.concatenate([zc, xo[:, :, 0:127]], axis=2)
    so0_ref[...] = jnp.concatenate([z1, xo_sh], axis=1)
    srcs = [so0_ref, se_ref, so_ref]
    taps = []
    for c in range(3):
        for kh in range(3):
            for kw in range(3):
                t = srcs[kw][c, pl.Slice(kh, 128, 2), :]
                taps.append(t.astype(jnp.bfloat16)[None])
    a3 = jnp.concatenate(taps, axis=0)
    x = jax.lax.dot_general(a3, w0_ref[...], (((0,), (0,)), ((), ())),
                            preferred_element_type=jnp.float32)
    x = _lrelu(x + b0_ref[...])
    s1_ref[...] = _pad_hw(x)

    x = _conv_taps2(s1_ref, w1_ref, b1_ref[...], 64, 64)
    x = _instnorm_lrelu(x)
    s2_ref[...] = _pad_hw(x.reshape(64, 64, 128))

    x = _conv_taps2(s2_ref, w2_ref, b2_ref[...], 32, 32)
    x = _instnorm_lrelu(x).astype(jnp.bfloat16)
    xp = _pad_hw(x.reshape(32, 32, 256))

    x = _conv_taps1(xp, w3_ref, b3_ref[...], 32, 32)
    x = _instnorm_lrelu(x).astype(jnp.bfloat16)
    xp = _pad_hw(x.reshape(32, 32, 512))

    o_ref[0] = _conv_taps1(xp, w4_ref, b4_ref[...], 32, 32)


def kernel(img, w0, b0, w1, b1, w2, b2, w3, b3, w4, b4):
    N = img.shape[0]
    imgi = jax.lax.bitcast_convert_type(
        img.astype(jnp.bfloat16).reshape(N, 3, 256, 128, 2), jnp.int32)

    w0m = jnp.transpose(w0, (2, 0, 1, 3)).reshape(27, 64).astype(jnp.bfloat16)
    b0m = b0.reshape(1, 64).astype(jnp.float32)
    w1m = w1.reshape(9, 64, 128).astype(jnp.bfloat16)
    b1m = b1.reshape(1, 128).astype(jnp.float32)
    w2m = w2.reshape(9, 128, 256).astype(jnp.bfloat16)
    b2m = b2.reshape(1, 256).astype(jnp.float32)
    w3m = w3.reshape(9, 256, 512).astype(jnp.bfloat16)
    b3m = b3.reshape(1, 512).astype(jnp.float32)
    w4m = jnp.pad(w4.reshape(9, 512, 1), ((0, 0), (0, 0), (0, 127)))
    w4m = w4m.astype(jnp.bfloat16)
    b4m = jnp.pad(b4.reshape(1, 1), ((0, 0), (0, 127))).astype(jnp.float32)

    out = pl.pallas_call(
        _disc_kernel,
        out_shape=jax.ShapeDtypeStruct((N, 1024, 128), jnp.float32),
        grid=(N,),
        in_specs=[
            pl.BlockSpec((1, 3, 256, 128), lambda n: (n, 0, 0, 0)),
            pl.BlockSpec((27, 64), lambda n: (0, 0)),
            pl.BlockSpec((1, 64), lambda n: (0, 0)),
            pl.BlockSpec((9, 64, 128), lambda n: (0, 0, 0)),
            pl.BlockSpec((1, 128), lambda n: (0, 0)),
            pl.BlockSpec((9, 128, 256), lambda n: (0, 0, 0)),
            pl.BlockSpec((1, 256), lambda n: (0, 0)),
            pl.BlockSpec((9, 256, 512), lambda n: (0, 0, 0)),
            pl.BlockSpec((1, 512), lambda n: (0, 0)),
            pl.BlockSpec((9, 512, 128), lambda n: (0, 0, 0)),
            pl.BlockSpec((1, 128), lambda n: (0, 0)),
        ],
        out_specs=pl.BlockSpec((1, 1024, 128), lambda n: (n, 0, 0)),
        scratch_shapes=[
            pltpu.VMEM((130, 130, 64), jnp.float32),
            pltpu.VMEM((66, 66, 128), jnp.float32),
            pltpu.VMEM((3, 257, 128), jnp.float32),
            pltpu.VMEM((3, 257, 128), jnp.float32),
            pltpu.VMEM((3, 257, 128), jnp.float32),
        ],
        compiler_params=pltpu.CompilerParams(
            dimension_semantics=("parallel",),
            vmem_limit_bytes=48 * 1024 * 1024,
        ),
    )(imgi, w0m, b0m, w1m, b1m, w2m, b2m, w3m, b3m, w4m, b4m)

    return out[:, :, 0].reshape(N, 1, 32, 32)


# speedup vs baseline: 27.5970x; 1.0044x over previous
import jax
import jax.numpy as jnp
from jax.experimental import pallas as pl
from jax.experimental.pallas import tpu as pltpu

_EPS = 1e-5
_SLOPE = 0.2


def _lrelu(x):
    return jnp.where(x >= 0.0, x, _SLOPE * x)


def _instnorm_lrelu(x):
    inv = 1.0 / x.shape[0]
    mean = jnp.sum(x, axis=0, keepdims=True) * inv
    xc = x - mean
    var = jnp.sum(xc * xc, axis=0, keepdims=True) * inv
    return _lrelu(xc * jax.lax.rsqrt(var + _EPS))


def _pad_hw(x):
    H, W, C = x.shape
    zc = jnp.zeros((H, 1, C), x.dtype)
    x = jnp.concatenate([zc, x, zc], axis=1)
    zr = jnp.zeros((1, W + 2, C), x.dtype)
    return jnp.concatenate([zr, x, zr], axis=0)


def _conv_taps1(xp, w_ref, bias, ho, wo):
    acc = bias
    for kh in range(3):
        for kw in range(3):
            a = xp[kh:kh + ho, kw:kw + wo, :].reshape(ho * wo, xp.shape[2])
            acc = acc + jnp.dot(a, w_ref[kh * 3 + kw],
                                preferred_element_type=jnp.float32)
    return acc


def _conv_taps2(s_ref, w_ref, bias, ho, wo):
    acc = bias
    c = s_ref.shape[2]
    for kh in range(3):
        for kw in range(3):
            a = s_ref[pl.Slice(kh, ho, 2), pl.Slice(kw, wo, 2), :]
            a = a.reshape(ho * wo, c).astype(jnp.bfloat16)
            acc = acc + jnp.dot(a, w_ref[kh * 3 + kw],
                                preferred_element_type=jnp.float32)
    return acc


def _disc_kernel(imgi_ref, w0_ref, b0_ref, w1_ref, b1_ref, w2_ref, b2_ref,
                 w3_ref, b3_ref, w4_ref, b4_ref, o_ref, s1_ref, s2_ref,
                 se_ref, so_ref, so0_ref):
    for g in range(2):
        _one_sample(imgi_ref.at[g], w0_ref, b0_ref, w1_ref, b1_ref, w2_ref,
                    b2_ref, w3_ref, b3_ref, w4_ref, b4_ref, o_ref.at[g],
                    s1_ref.at[g], s2_ref.at[g], se_ref.at[g], so_ref.at[g],
                    so0_ref.at[g])


def _one_sample(imgi_ref, w0_ref, b0_ref, w1_ref, b1_ref, w2_ref, b2_ref,
                w3_ref, b3_ref, w4_ref, b4_ref, o_ref, s1_ref, s2_ref,
                se_ref, so_ref, so0_ref):
    v = imgi_ref[...]
    xe = pltpu.unpack_elementwise(v, index=0, packed_dtype=jnp.bfloat16,
                                  unpacked_dtype=jnp.float32)
    xo = pltpu.unpack_elementwise(v, index=1, packed_dtype=jnp.bfloat16,
                                  unpacked_dtype=jnp.float32)
    z1 = jnp.zeros((3, 1, 128), jnp.float32)
    se_ref[...] = jnp.concatenate([z1, xe], axis=1)
    so_ref[...] = jnp.concatenate([z1, xo], axis=1)
    zc = jnp.zeros((3, 256, 1), jnp.float32)
    xo_sh = jnp.concatenate([zc, xo[:, :, 0:127]], axis=2)
    so0_ref[...] = jnp.concatenate([z1, xo_sh], axis=1)
    srcs = [so0_ref, se_ref, so_ref]
    taps = []
    for c in range(3):
        for kh in range(3):
            for kw in range(3):
                t = srcs[kw][c, pl.Slice(kh, 128, 2), :]
                taps.append(t.astype(jnp.bfloat16)[None])
    a3 = jnp.concatenate(taps, axis=0)
    x = jax.lax.dot_general(a3, w0_ref[...], (((0,), (0,)), ((), ())),
                            preferred_element_type=jnp.float32)
    x = _lrelu(x + b0_ref[...])
    s1_ref[...] = _pad_hw(x)

    x = _conv_taps2(s1_ref, w1_ref, b1_ref[...], 64, 64)
    x = _instnorm_lrelu(x)
    s2_ref[...] = _pad_hw(x.reshape(64, 64, 128))

    x = _conv_taps2(s2_ref, w2_ref, b2_ref[...], 32, 32)
    x = _instnorm_lrelu(x).astype(jnp.bfloat16)
    xp = _pad_hw(x.reshape(32, 32, 256))

    x = _conv_taps1(xp, w3_ref, b3_ref[...], 32, 32)
    x = _instnorm_lrelu(x).astype(jnp.bfloat16)
    xp = _pad_hw(x.reshape(32, 32, 512))

    o_ref[...] = _conv_taps1(xp, w4_ref, b4_ref[...], 32, 32)


def kernel(img, w0, b0, w1, b1, w2, b2, w3, b3, w4, b4):
    N = img.shape[0]
    imgi = jax.lax.bitcast_convert_type(
        img.astype(jnp.bfloat16).reshape(N, 3, 256, 128, 2), jnp.int32)

    w0m = jnp.transpose(w0, (2, 0, 1, 3)).reshape(27, 64).astype(jnp.bfloat16)
    b0m = b0.reshape(1, 64).astype(jnp.float32)
    w1m = w1.reshape(9, 64, 128).astype(jnp.bfloat16)
    b1m = b1.reshape(1, 128).astype(jnp.float32)
    w2m = w2.reshape(9, 128, 256).astype(jnp.bfloat16)
    b2m = b2.reshape(1, 256).astype(jnp.float32)
    w3m = w3.reshape(9, 256, 512).astype(jnp.bfloat16)
    b3m = b3.reshape(1, 512).astype(jnp.float32)
    w4m = jnp.pad(w4.reshape(9, 512, 1), ((0, 0), (0, 0), (0, 127)))
    w4m = w4m.astype(jnp.bfloat16)
    b4m = jnp.pad(b4.reshape(1, 1), ((0, 0), (0, 127))).astype(jnp.float32)

    out = pl.pallas_call(
        _disc_kernel,
        out_shape=jax.ShapeDtypeStruct((N, 1024, 128), jnp.float32),
        grid=(N // 2,),
        in_specs=[
            pl.BlockSpec((2, 3, 256, 128), lambda i: (i, 0, 0, 0)),
            pl.BlockSpec((27, 64), lambda i: (0, 0)),
            pl.BlockSpec((1, 64), lambda i: (0, 0)),
            pl.BlockSpec((9, 64, 128), lambda i: (0, 0, 0)),
            pl.BlockSpec((1, 128), lambda i: (0, 0)),
            pl.BlockSpec((9, 128, 256), lambda i: (0, 0, 0)),
            pl.BlockSpec((1, 256), lambda i: (0, 0)),
            pl.BlockSpec((9, 256, 512), lambda i: (0, 0, 0)),
            pl.BlockSpec((1, 512), lambda i: (0, 0)),
            pl.BlockSpec((9, 512, 128), lambda i: (0, 0, 0)),
            pl.BlockSpec((1, 128), lambda i: (0, 0)),
        ],
        out_specs=pl.BlockSpec((2, 1024, 128), lambda i: (i, 0, 0)),
        scratch_shapes=[
            pltpu.VMEM((2, 130, 130, 64), jnp.float32),
            pltpu.VMEM((2, 66, 66, 128), jnp.float32),
            pltpu.VMEM((2, 3, 257, 128), jnp.float32),
            pltpu.VMEM((2, 3, 257, 128), jnp.float32),
            pltpu.VMEM((2, 3, 257, 128), jnp.float32),
        ],
        compiler_params=pltpu.CompilerParams(
            dimension_semantics=("arbitrary",),
            vmem_limit_bytes=48 * 1024 * 1024,
        ),
    )(imgi, w0m, b0m, w1m, b1m, w2m, b2m, w3m, b3m, w4m, b4m)

    return out[:, :, 0].reshape(N, 1, 32, 32)
